# baseline (speedup 1.0000x reference)
import numpy as np

import jax
import jax.numpy as jnp
from jax import lax
from jax.experimental import pallas as pl
from jax.experimental.pallas import tpu as pltpu
from jax.experimental.pallas import tpu_sc as plsc

B = 1024
V = 100000
KS = (1, 5, 10, 20)

NC = 2
NS = 16
NW = NC * NS
L = 16
LG = B // L

VSPLIT = 63136

IPW = (V - VSPLIT) // NW
CI = 24
NCHUNK = IPW // CI
NBUF = 4

TC_BR = 512
TC_GRID = -(-VSPLIT // TC_BR)
TC_LAST_SLABS = (VSPLIT - (TC_GRID - 1) * TC_BR) // 8

NMET = 9

_mesh = plsc.VectorSubcoreMesh(core_axis_name="c", subcore_axis_name="s")
_sc_params = pltpu.CompilerParams(needs_layout_passes=False)


def _worker_id():
    return lax.axis_index("s") * NC + lax.axis_index("c")


PQW = 32


def _pred_body(scores_hbm, labels_hbm, o0, o1, o2, o3,
               lab_ref, pblk_ref, stage_ref, gsem):
    wid = _worker_id()
    lane = lax.iota(jnp.int32, L)
    q = wid % 4
    base_b = pl.multiple_of((wid // 4) * 128, 128)
    base_l = pl.multiple_of(wid * PQW, PQW)

    pltpu.sync_copy(labels_hbm.at[pl.ds(base_l, PQW)], lab_ref)
    for g in range(PQW // L):
        lab_v = lab_ref[pl.ds(g * L, L)]
        for r8 in range(L):
            lab = lab_v[r8]
            row0 = pl.multiple_of((lab // 8) * 8, 8)
            pltpu.async_copy(
                scores_hbm.at[pl.ds(row0, 8), pl.ds(base_b, 128)],
                pblk_ref.at[g * L + r8], gsem)
    zf = jnp.zeros((L,), jnp.float32)
    for r in range(8):
        for c in range(128 // L):
            stage_ref[r, pl.ds(c * L, L)] = zf
    for k in range(PQW):
        pltpu.make_async_copy(scores_hbm.at[pl.ds(0, 8), pl.ds(0, 128)],
                              pblk_ref.at[0], gsem).wait()
    for g in range(PQW // L):
        lab_v = lab_ref[pl.ds(g * L, L)]
        k_v = g * L + lane
        srow_v = lab_v - (lab_v // 8) * 8
        scol_v = q * PQW + g * L + lane
        p_v = plsc.load_gather(pblk_ref, [k_v, srow_v, scol_v])
        o = pl.multiple_of(q * PQW + g * L, L)
        for r in range(8):
            stage_ref[r, pl.ds(o, L)] = p_v
    outs = (o0, o1, o2, o3)
    for qq in range(4):
        @pl.when(q == qq)
        def _(qq=qq):
            pltpu.sync_copy(stage_ref,
                            outs[qq].at[pl.ds(0, 8), pl.ds(base_b, 128)])


_pred_call = pl.kernel(
    _pred_body,
    out_type=[jax.ShapeDtypeStruct((8, B), jnp.float32)] * 4,
    mesh=_mesh,
    compiler_params=_sc_params,
    scratch_types=[
        pltpu.VMEM((PQW,), jnp.int32),
        pltpu.VMEM((PQW, 8, 128), jnp.float32),
        pltpu.VMEM((8, 128), jnp.float32),
        pltpu.SemaphoreType.DMA,
    ],
)


def _tc_count_body(p0_ref, p1_ref, p2_ref, p3_ref, scores_ref, out_ref,
                   acc_ref):
    j = pl.program_id(0)

    @pl.when(j == 0)
    def _():
        acc_ref[...] = jnp.zeros_like(acc_ref)

    p = (p0_ref[...] + p1_ref[...]) + (p2_ref[...] + p3_ref[...])

    def accumulate(n_slabs):
        x = scores_ref[...]
        a = acc_ref[...]
        for s in range(n_slabs):
            a = a + (x[s * 8:(s + 1) * 8, :] > p).astype(jnp.float32)
        acc_ref[...] = a

    @pl.when(j < TC_GRID - 1)
    def _():
        accumulate(TC_BR // 8)

    @pl.when(j == TC_GRID - 1)
    def _():
        accumulate(TC_LAST_SLABS)

    @pl.when(j == TC_GRID - 1)
    def _():
        out_ref[...] = acc_ref[...]


_tc_count = pl.pallas_call(
    _tc_count_body,
    grid=(TC_GRID,),
    in_specs=[
        pl.BlockSpec((8, B), lambda j: (0, 0)),
        pl.BlockSpec((8, B), lambda j: (0, 0)),
        pl.BlockSpec((8, B), lambda j: (0, 0)),
        pl.BlockSpec((8, B), lambda j: (0, 0)),
        pl.BlockSpec((TC_BR, B), lambda j: (j, 0)),
    ],
    out_specs=pl.BlockSpec((8, B), lambda j: (0, 0)),
    out_shape=jax.ShapeDtypeStruct((8, B), jnp.float32),
    scratch_shapes=[pltpu.VMEM((8, B), jnp.float32)],
    compiler_params=pltpu.CompilerParams(
        dimension_semantics=("arbitrary",)),
)


def _scan_body(scores_hbm, p0_hbm, p1_hbm, p2_hbm, p3_hbm, out_hbm,
               pred_ref, acc_ref, stage_ref,
               b0, b1, b2, b3, s0, s1, s2, s3, gsem):
    bufs = (b0, b1, b2, b3)
    sems = (s0, s1, s2, s3)
    wid = _worker_id()
    item0 = pl.multiple_of(VSPLIT + wid * IPW, 8)

    pltpu.sync_copy(p0_hbm, b0.at[pl.ds(0, 8)])
    pltpu.sync_copy(p1_hbm, b0.at[pl.ds(8, 8)])
    pltpu.sync_copy(p2_hbm, b0.at[pl.ds(16, 8)])
    pltpu.sync_copy(p3_hbm, b1.at[pl.ds(0, 8)])

    @pl.loop(0, LG)
    def _(lg):
        o = lg * L
        pred_ref[pl.ds(o, L)] = ((b0[0, pl.ds(o, L)] + b0[8, pl.ds(o, L)])
                                 + (b0[16, pl.ds(o, L)] + b1[0, pl.ds(o, L)]))

    def issue_chunk(t, b):
        src = pl.multiple_of(item0 + t * CI, 8)
        pltpu.async_copy(scores_hbm.at[pl.ds(src, CI)], bufs[b], sems[b])

    for b in range(NBUF):
        issue_chunk(b, b)

    zf = jnp.zeros((L,), jnp.float32)

    @pl.loop(0, LG)
    def _(lg):
        acc_ref[pl.ds(lg * L, L)] = zf

    @pl.loop(0, NCHUNK, step=NBUF)
    def _(t_base):
        for b in range(NBUF):
            t = t_base + b
            buf, sem = bufs[b], sems[b]
            pltpu.make_async_copy(scores_hbm.at[pl.ds(0, CI)],
                                  buf, sem).wait()

            def body(lg, _):
                o = lg * L
                pred_v = pred_ref[pl.ds(o, L)]
                acc_v = acc_ref[pl.ds(o, L)]
                for it in range(CI):
                    x = buf[it, pl.ds(o, L)]
                    acc_v = acc_v + (x > pred_v).astype(jnp.float32)
                acc_ref[pl.ds(o, L)] = acc_v
                return 0

            lax.fori_loop(0, LG, body, 0)

            @pl.when(t + NBUF < NCHUNK)
            def _():
                issue_chunk(t + NBUF, b)

    @pl.loop(0, LG)
    def _(lg):
        o = lg * L
        stage_ref[0, pl.ds(o, L)] = acc_ref[pl.ds(o, L)]
        for r in range(1, 8):
            stage_ref[r, pl.ds(o, L)] = zf

    row_out = pl.multiple_of(wid * 8, 8)
    pltpu.sync_copy(stage_ref, out_hbm.at[pl.ds(row_out, 8)])


_scan_call = pl.kernel(
    _scan_body,
    out_type=jax.ShapeDtypeStruct((NW * 8, B), jnp.float32),
    mesh=_mesh,
    compiler_params=_sc_params,
    scratch_types=[
        pltpu.VMEM((B,), jnp.float32),
        pltpu.VMEM((B,), jnp.float32),
        pltpu.VMEM((8, B), jnp.float32),
        pltpu.VMEM((CI, B), jnp.float32),
        pltpu.VMEM((CI, B), jnp.float32),
        pltpu.VMEM((CI, B), jnp.float32),
        pltpu.VMEM((CI, B), jnp.float32),
        pltpu.SemaphoreType.DMA,
        pltpu.SemaphoreType.DMA,
        pltpu.SemaphoreType.DMA,
        pltpu.SemaphoreType.DMA,
        pltpu.SemaphoreType.DMA,
    ],
)


def _combine_body(tc_ref, sc_ref, out_ref):
    rank = (jnp.sum(tc_ref[...], axis=0, keepdims=True)
            + jnp.sum(sc_ref[...], axis=0, keepdims=True))
    inv_b = 1.0 / B
    dcg = np.float32(np.log(2.0)) / jnp.log(rank + 2.0)
    res = []
    for k in KS:
        ind = (rank < float(k)).astype(jnp.float32)
        res.append(jnp.sum(dcg * ind) * inv_b)
        res.append(jnp.sum(ind) * inv_b)
    res.append(jnp.sum(1.0 / (rank + 1.0)) * inv_b)
    out_ref[...] = jnp.stack(res)


_tc_combine = pl.pallas_call(
    _combine_body,
    in_specs=[
        pl.BlockSpec((8, B), lambda: (0, 0)),
        pl.BlockSpec((NW * 8, B), lambda: (0, 0)),
    ],
    out_specs=pl.BlockSpec((NMET,), lambda: (0,)),
    out_shape=jax.ShapeDtypeStruct((NMET,), jnp.float32),
)


def kernel(scores, labels):
    scores_t = scores.T
    p0, p1, p2, p3 = _pred_call(scores_t, labels)
    tc8 = _tc_count(p0, p1, p2, p3, scores_t)
    sc256 = _scan_call(scores_t, p0, p1, p2, p3)
    return _tc_combine(tc8, sc256)


# speedup vs baseline: 1.1054x; 1.1054x over previous
import numpy as np

import jax
import jax.numpy as jnp
from jax import lax
from jax.experimental import pallas as pl
from jax.experimental.pallas import tpu as pltpu
from jax.experimental.pallas import tpu_sc as plsc

B = 1024
V = 100000
KS = (1, 5, 10, 20)

NC = 2
NS = 16
NW = NC * NS
L = 16
LG = B // L

VSPLIT = 63136

IPW = (V - VSPLIT) // NW
CI = 24
NCHUNK = IPW // CI
NBUF = 4

TC_BR = 2048
TC_GRID = -(-VSPLIT // TC_BR)
TC_LAST_SLABS = (VSPLIT - (TC_GRID - 1) * TC_BR) // 8

NMET = 9

_mesh = plsc.VectorSubcoreMesh(core_axis_name="c", subcore_axis_name="s")
_sc_params = pltpu.CompilerParams(needs_layout_passes=False)


def _worker_id():
    return lax.axis_index("s") * NC + lax.axis_index("c")


PQW = 32


def _pred_body(scores_hbm, labels_hbm, o0, o1, o2, o3,
               lab_ref, pblk_ref, stage_ref, gsem):
    wid = _worker_id()
    lane = lax.iota(jnp.int32, L)
    q = wid % 4
    base_b = pl.multiple_of((wid // 4) * 128, 128)
    base_l = pl.multiple_of(wid * PQW, PQW)

    pltpu.sync_copy(labels_hbm.at[pl.ds(base_l, PQW)], lab_ref)
    for g in range(PQW // L):
        lab_v = lab_ref[pl.ds(g * L, L)]
        for r8 in range(L):
            lab = lab_v[r8]
            row0 = pl.multiple_of((lab // 8) * 8, 8)
            pltpu.async_copy(
                scores_hbm.at[pl.ds(row0, 8), pl.ds(base_b, 128)],
                pblk_ref.at[g * L + r8], gsem)
    zf = jnp.zeros((L,), jnp.float32)
    for r in range(8):
        for c in range(128 // L):
            stage_ref[r, pl.ds(c * L, L)] = zf
    for k in range(PQW):
        pltpu.make_async_copy(scores_hbm.at[pl.ds(0, 8), pl.ds(0, 128)],
                              pblk_ref.at[0], gsem).wait()
    for g in range(PQW // L):
        lab_v = lab_ref[pl.ds(g * L, L)]
        k_v = g * L + lane
        srow_v = lab_v - (lab_v // 8) * 8
        scol_v = q * PQW + g * L + lane
        p_v = plsc.load_gather(pblk_ref, [k_v, srow_v, scol_v])
        o = pl.multiple_of(q * PQW + g * L, L)
        for r in range(8):
            stage_ref[r, pl.ds(o, L)] = p_v
    outs = (o0, o1, o2, o3)
    for qq in range(4):
        @pl.when(q == qq)
        def _(qq=qq):
            pltpu.sync_copy(stage_ref,
                            outs[qq].at[pl.ds(0, 8), pl.ds(base_b, 128)])


_pred_call = pl.kernel(
    _pred_body,
    out_type=[jax.ShapeDtypeStruct((8, B), jnp.float32)] * 4,
    mesh=_mesh,
    compiler_params=_sc_params,
    scratch_types=[
        pltpu.VMEM((PQW,), jnp.int32),
        pltpu.VMEM((PQW, 8, 128), jnp.float32),
        pltpu.VMEM((8, 128), jnp.float32),
        pltpu.SemaphoreType.DMA,
    ],
)


def _tc_count_body(p0_ref, p1_ref, p2_ref, p3_ref, scores_ref, out_ref,
                   acc_ref):
    j = pl.program_id(0)

    @pl.when(j == 0)
    def _():
        acc_ref[...] = jnp.zeros_like(acc_ref)

    p = (p0_ref[...] + p1_ref[...]) + (p2_ref[...] + p3_ref[...])

    def accumulate(n_slabs):
        x = scores_ref[...]
        a = acc_ref[...]
        for s in range(n_slabs):
            a = a + (x[s * 8:(s + 1) * 8, :] > p).astype(jnp.float32)
        acc_ref[...] = a

    @pl.when(j < TC_GRID - 1)
    def _():
        accumulate(TC_BR // 8)

    @pl.when(j == TC_GRID - 1)
    def _():
        accumulate(TC_LAST_SLABS)

    @pl.when(j == TC_GRID - 1)
    def _():
        out_ref[...] = acc_ref[...]


_tc_count = pl.pallas_call(
    _tc_count_body,
    grid=(TC_GRID,),
    in_specs=[
        pl.BlockSpec((8, B), lambda j: (0, 0)),
        pl.BlockSpec((8, B), lambda j: (0, 0)),
        pl.BlockSpec((8, B), lambda j: (0, 0)),
        pl.BlockSpec((8, B), lambda j: (0, 0)),
        pl.BlockSpec((TC_BR, B), lambda j: (j, 0)),
    ],
    out_specs=pl.BlockSpec((8, B), lambda j: (0, 0)),
    out_shape=jax.ShapeDtypeStruct((8, B), jnp.float32),
    scratch_shapes=[pltpu.VMEM((8, B), jnp.float32)],
    compiler_params=pltpu.CompilerParams(
        dimension_semantics=("arbitrary",)),
)


def _scan_body(scores_hbm, p0_hbm, p1_hbm, p2_hbm, p3_hbm, out_hbm,
               pred_ref, acc_ref, stage_ref,
               b0, b1, b2, b3, s0, s1, s2, s3, gsem):
    bufs = (b0, b1, b2, b3)
    sems = (s0, s1, s2, s3)
    wid = _worker_id()
    item0 = pl.multiple_of(VSPLIT + wid * IPW, 8)

    pltpu.sync_copy(p0_hbm, b0.at[pl.ds(0, 8)])
    pltpu.sync_copy(p1_hbm, b0.at[pl.ds(8, 8)])
    pltpu.sync_copy(p2_hbm, b0.at[pl.ds(16, 8)])
    pltpu.sync_copy(p3_hbm, b1.at[pl.ds(0, 8)])

    @pl.loop(0, LG)
    def _(lg):
        o = lg * L
        pred_ref[pl.ds(o, L)] = ((b0[0, pl.ds(o, L)] + b0[8, pl.ds(o, L)])
                                 + (b0[16, pl.ds(o, L)] + b1[0, pl.ds(o, L)]))

    def issue_chunk(t, b):
        src = pl.multiple_of(item0 + t * CI, 8)
        pltpu.async_copy(scores_hbm.at[pl.ds(src, CI)], bufs[b], sems[b])

    for b in range(NBUF):
        issue_chunk(b, b)

    zf = jnp.zeros((L,), jnp.float32)

    @pl.loop(0, LG)
    def _(lg):
        acc_ref[pl.ds(lg * L, L)] = zf

    @pl.loop(0, NCHUNK, step=NBUF)
    def _(t_base):
        for b in range(NBUF):
            t = t_base + b
            buf, sem = bufs[b], sems[b]
            pltpu.make_async_copy(scores_hbm.at[pl.ds(0, CI)],
                                  buf, sem).wait()

            def body(lg, _):
                o = lg * L
                pred_v = pred_ref[pl.ds(o, L)]
                acc_v = acc_ref[pl.ds(o, L)]
                for it in range(CI):
                    x = buf[it, pl.ds(o, L)]
                    acc_v = acc_v + (x > pred_v).astype(jnp.float32)
                acc_ref[pl.ds(o, L)] = acc_v
                return 0

            lax.fori_loop(0, LG, body, 0)

            @pl.when(t + NBUF < NCHUNK)
            def _():
                issue_chunk(t + NBUF, b)

    @pl.loop(0, LG)
    def _(lg):
        o = lg * L
        stage_ref[0, pl.ds(o, L)] = acc_ref[pl.ds(o, L)]
        for r in range(1, 8):
            stage_ref[r, pl.ds(o, L)] = zf

    row_out = pl.multiple_of(wid * 8, 8)
    pltpu.sync_copy(stage_ref, out_hbm.at[pl.ds(row_out, 8)])


_scan_call = pl.kernel(
    _scan_body,
    out_type=jax.ShapeDtypeStruct((NW * 8, B), jnp.float32),
    mesh=_mesh,
    compiler_params=_sc_params,
    scratch_types=[
        pltpu.VMEM((B,), jnp.float32),
        pltpu.VMEM((B,), jnp.float32),
        pltpu.VMEM((8, B), jnp.float32),
        pltpu.VMEM((CI, B), jnp.float32),
        pltpu.VMEM((CI, B), jnp.float32),
        pltpu.VMEM((CI, B), jnp.float32),
        pltpu.VMEM((CI, B), jnp.float32),
        pltpu.SemaphoreType.DMA,
        pltpu.SemaphoreType.DMA,
        pltpu.SemaphoreType.DMA,
        pltpu.SemaphoreType.DMA,
        pltpu.SemaphoreType.DMA,
    ],
)


def _combine_body(tc_ref, sc_ref, out_ref):
    rank = (jnp.sum(tc_ref[...], axis=0, keepdims=True)
            + jnp.sum(sc_ref[...], axis=0, keepdims=True))
    inv_b = 1.0 / B
    dcg = np.float32(np.log(2.0)) / jnp.log(rank + 2.0)
    res = []
    for k in KS:
        ind = (rank < float(k)).astype(jnp.float32)
        res.append(jnp.sum(dcg * ind) * inv_b)
        res.append(jnp.sum(ind) * inv_b)
    res.append(jnp.sum(1.0 / (rank + 1.0)) * inv_b)
    out_ref[...] = jnp.stack(res)


_tc_combine = pl.pallas_call(
    _combine_body,
    in_specs=[
        pl.BlockSpec((8, B), lambda: (0, 0)),
        pl.BlockSpec((NW * 8, B), lambda: (0, 0)),
    ],
    out_specs=pl.BlockSpec((NMET,), lambda: (0,)),
    out_shape=jax.ShapeDtypeStruct((NMET,), jnp.float32),
)


def kernel(scores, labels):
    scores_t = scores.T
    p0, p1, p2, p3 = _pred_call(scores_t, labels)
    tc8 = _tc_count(p0, p1, p2, p3, scores_t)
    sc256 = _scan_call(scores_t, p0, p1, p2, p3)
    return _tc_combine(tc8, sc256)
